# baseline (speedup 1.0000x reference)
import functools

import jax
import jax.numpy as jnp
from jax import lax
from jax.experimental import pallas as pl
from jax.experimental.pallas import tpu as pltpu
from jax.experimental.pallas import tpu_sc as plsc

B, L, E, DIN, DM = 1024, 16, 8, 900, 768
EPS = 1e-9
TS = 32
NT = (2 * B + E * (TS - 1)) // TS + 1
P = NT * TS
A = 2 * B


def _routing_body(logits_ref, masks_ref, gates_ref, pos_ref, te_ref):
    logits = logits_ref[...]
    mask = jnp.where(masks_ref[...] == 1.0, 1.0, 0.0)
    m = jnp.max(logits, axis=1, keepdims=True)
    ex = jnp.exp(logits - m)
    probs = ex / jnp.sum(ex, axis=1, keepdims=True)
    g = probs * mask
    lane = lax.broadcasted_iota(jnp.int32, (B, E), 1)
    m1 = jnp.max(g, axis=1, keepdims=True)
    i1 = jnp.min(jnp.where(g == m1, lane, E), axis=1, keepdims=True)
    g_wo = jnp.where(lane == i1, -jnp.inf, g)
    m2 = jnp.max(g_wo, axis=1, keepdims=True)
    i2 = jnp.min(jnp.where(g_wo == m2, lane, E), axis=1, keepdims=True)
    denom = m1 + m2 + EPS
    gates_ref[...] = jnp.concatenate([m1, m2], axis=1) / denom

    sel1 = lane == i1
    sel2 = lane == i2
    assign = jnp.where(sel1 | sel2, 1.0, 0.0)
    row = lax.broadcasted_iota(jnp.int32, (B, B), 0)
    col = lax.broadcasted_iota(jnp.int32, (B, B), 1)
    tlow = jnp.where(row > col, 1.0, 0.0)
    ranks = jnp.dot(tlow, assign, preferred_element_type=jnp.float32)
    counts = jnp.sum(assign, axis=0, keepdims=True)
    tiles_e = jnp.floor((counts + (TS - 1)) / TS)
    r8 = lax.broadcasted_iota(jnp.int32, (E, E), 0)
    c8 = lax.broadcasted_iota(jnp.int32, (E, E), 1)
    incl = jnp.dot(tiles_e, jnp.where(r8 <= c8, 1.0, 0.0),
                   preferred_element_type=jnp.float32)
    excl = incl - tiles_e
    pos_dense = TS * excl + ranks
    pos1 = jnp.sum(jnp.where(sel1, pos_dense, 0.0), axis=1, keepdims=True)
    pos2 = jnp.sum(jnp.where(sel2, pos_dense, 0.0), axis=1, keepdims=True)
    pos_ref[...] = jnp.concatenate([pos1, pos2], axis=1).astype(jnp.int32)

    tt = lax.broadcasted_iota(jnp.int32, (NT, E), 0).astype(jnp.float32)
    te = jnp.sum(jnp.where(jnp.broadcast_to(incl, (NT, E)) <= tt, 1.0, 0.0),
                 axis=1, keepdims=True)
    te_ref[...] = jnp.minimum(te, E - 1).astype(jnp.int32)


def _routing(logits, moe_masks):
    return pl.pallas_call(
        _routing_body,
        out_shape=(
            jax.ShapeDtypeStruct((B, 2), jnp.float32),
            jax.ShapeDtypeStruct((B, 2), jnp.int32),
            jax.ShapeDtypeStruct((NT, 1), jnp.int32),
        ),
    )(logits, moe_masks)


def _scatter_body(pos_hbm, rs_hbm, pos_v, rs_v, sem):
    pltpu.async_copy(pos_hbm, pos_v, sem).wait()
    zero = jnp.zeros((16,), jnp.int32)
    for i in range(P // 16):
        rs_v[pl.ds(i * 16, 16)] = zero
    for i in range(A // 16):
        idx = pos_v[pl.ds(i * 16, 16)]
        ids = (i * 16 + lax.iota(jnp.int32, 16)) >> 1
        plsc.store_scatter(rs_v, [idx], ids)
    nsub = 32
    wid = lax.axis_index("s") * 2 + lax.axis_index("c")
    per = P // nsub
    base = wid * per
    pltpu.sync_copy(rs_v.at[pl.ds(base, per)], rs_hbm.at[pl.ds(base, per)])


def _scatter(pos_flat):
    mesh = plsc.VectorSubcoreMesh(core_axis_name="c", subcore_axis_name="s")
    fn = functools.partial(
        pl.kernel,
        out_type=jax.ShapeDtypeStruct((P,), jnp.int32),
        mesh=mesh,
        scratch_types=[
            pltpu.VMEM((A,), jnp.int32),
            pltpu.VMEM((P,), jnp.int32),
            pltpu.SemaphoreType.DMA,
        ],
        compiler_params=pltpu.CompilerParams(needs_layout_passes=False),
    )(_scatter_body)
    return fn(pos_flat)


def _matmul_body(rs_ref, te_ref, x_ref, w_ref, b_ref, y_ref, xbuf, sems):
    t = pl.program_id(0)

    def issue(tile, buf):
        for j in range(TS):
            pltpu.make_async_copy(
                x_ref.at[rs_ref[tile * TS + j]],
                xbuf.at[buf, pl.ds(j * L, L), :],
                sems.at[buf],
            ).start()

    def drain(tile, buf):
        for j in range(TS):
            pltpu.make_async_copy(
                x_ref.at[rs_ref[tile * TS + j]],
                xbuf.at[buf, pl.ds(j * L, L), :],
                sems.at[buf],
            ).wait()

    @pl.when(t == 0)
    def _():
        issue(0, 0)

    @pl.when(t + 1 < NT)
    def _():
        issue(t + 1, (t + 1) % 2)

    drain(t, t % 2)
    x = xbuf[t % 2]
    y = jnp.dot(x, w_ref[0], preferred_element_type=jnp.float32)
    y_ref[0] = y + b_ref[0]


def _matmul(x, W, b, row_sample, tile_expert):
    grid_spec = pltpu.PrefetchScalarGridSpec(
        num_scalar_prefetch=2,
        grid=(NT,),
        in_specs=[
            pl.BlockSpec(memory_space=pl.ANY),
            pl.BlockSpec((1, DIN, DM), lambda t, rs, te: (te[t], 0, 0)),
            pl.BlockSpec((1, 1, DM), lambda t, rs, te: (te[t], 0, 0)),
        ],
        out_specs=pl.BlockSpec((1, TS * L, DM), lambda t, rs, te: (t, 0, 0)),
        scratch_shapes=[
            pltpu.VMEM((2, TS * L, DIN), jnp.float32),
            pltpu.SemaphoreType.DMA((2,)),
        ],
    )
    return pl.pallas_call(
        _matmul_body,
        grid_spec=grid_spec,
        out_shape=jax.ShapeDtypeStruct((NT, TS * L, DM), jnp.float32),
    )(row_sample, tile_expert, x, W, b.reshape(E, 1, DM))


_CS = 8


def _combine_body(p1_ref, p2_ref, g_ref, *refs):
    i = pl.program_id(0)
    y1 = refs[:_CS]
    y2 = refs[_CS:2 * _CS]
    out_ref = refs[2 * _CS]
    for j in range(_CS):
        s = i * _CS + j
        o = g_ref[s, 0] * y1[j][0] + g_ref[s, 1] * y2[j][0]
        out_ref[j] = o.astype(jnp.bfloat16)


def _combine(Y, pos1, pos2, gates):
    def spec1(j):
        return pl.BlockSpec((1, L, DM),
                            lambda i, p1, p2, g, j=j: (p1[i * _CS + j], 0, 0))

    def spec2(j):
        return pl.BlockSpec((1, L, DM),
                            lambda i, p1, p2, g, j=j: (p2[i * _CS + j], 0, 0))

    grid_spec = pltpu.PrefetchScalarGridSpec(
        num_scalar_prefetch=3,
        grid=(B // _CS,),
        in_specs=[spec1(j) for j in range(_CS)] + [spec2(j) for j in range(_CS)],
        out_specs=pl.BlockSpec((_CS, L, DM), lambda i, p1, p2, g: (i, 0, 0)),
    )
    return pl.pallas_call(
        _combine_body,
        grid_spec=grid_spec,
        out_shape=jax.ShapeDtypeStruct((B, L, DM), jnp.bfloat16),
    )(pos1, pos2, gates, *([Y] * (2 * _CS)))


def kernel(cycle_curve_data, logits, moe_masks, W, b):
    gates, pos, tile_expert = _routing(logits, moe_masks)
    row_sample = _scatter(pos.reshape(A))
    Y = _matmul(cycle_curve_data, W, b, row_sample, tile_expert.reshape(NT))
    Yv = Y.reshape(P, L, DM)
    out = _combine(Yv, pos[:, 0], pos[:, 1], gates)
    return out


# speedup vs baseline: 1.4452x; 1.4452x over previous
import functools

import jax
import jax.numpy as jnp
from jax import lax
from jax.experimental import pallas as pl
from jax.experimental.pallas import tpu as pltpu
from jax.experimental.pallas import tpu_sc as plsc

B, L, E, DIN, DM = 1024, 16, 8, 900, 768
EPS = 1e-9
TS = 32
NT = (2 * B + E * (TS - 1)) // TS + 1
P = NT * TS
A = 2 * B
OROWS = 2 * (B + 1)
CH = 16


def _routing_body(logits_ref, masks_ref, gates_ref, pos_ref, te_ref):
    logits = logits_ref[...]
    mask = jnp.where(masks_ref[...] == 1.0, 1.0, 0.0)
    m = jnp.max(logits, axis=1, keepdims=True)
    ex = jnp.exp(logits - m)
    probs = ex / jnp.sum(ex, axis=1, keepdims=True)
    g = probs * mask
    lane = lax.broadcasted_iota(jnp.int32, (B, E), 1)
    m1 = jnp.max(g, axis=1, keepdims=True)
    i1 = jnp.min(jnp.where(g == m1, lane, E), axis=1, keepdims=True)
    g_wo = jnp.where(lane == i1, -jnp.inf, g)
    m2 = jnp.max(g_wo, axis=1, keepdims=True)
    i2 = jnp.min(jnp.where(g_wo == m2, lane, E), axis=1, keepdims=True)
    denom = m1 + m2 + EPS
    gates_ref[...] = jnp.concatenate([m1, m2], axis=1) / denom

    sel1 = lane == i1
    sel2 = lane == i2
    assign = jnp.where(sel1 | sel2, 1.0, 0.0)
    row = lax.broadcasted_iota(jnp.int32, (B, B), 0)
    col = lax.broadcasted_iota(jnp.int32, (B, B), 1)
    tlow = jnp.where(row > col, 1.0, 0.0)
    ranks = jnp.dot(tlow, assign, preferred_element_type=jnp.float32)
    counts = jnp.sum(assign, axis=0, keepdims=True)
    tiles_e = jnp.floor((counts + (TS - 1)) / TS)
    r8 = lax.broadcasted_iota(jnp.int32, (E, E), 0)
    c8 = lax.broadcasted_iota(jnp.int32, (E, E), 1)
    incl = jnp.dot(tiles_e, jnp.where(r8 <= c8, 1.0, 0.0),
                   preferred_element_type=jnp.float32)
    excl = incl - tiles_e
    pos_dense = TS * excl + ranks
    pos1 = jnp.sum(jnp.where(sel1, pos_dense, 0.0), axis=1, keepdims=True)
    pos2 = jnp.sum(jnp.where(sel2, pos_dense, 0.0), axis=1, keepdims=True)
    pos_ref[...] = jnp.concatenate([pos1, pos2], axis=1).astype(jnp.int32)

    tt = lax.broadcasted_iota(jnp.int32, (NT, E), 0).astype(jnp.float32)
    te = jnp.sum(jnp.where(jnp.broadcast_to(incl, (NT, E)) <= tt, 1.0, 0.0),
                 axis=1, keepdims=True)
    te_ref[...] = jnp.minimum(te, E - 1).astype(jnp.int32)


def _routing(logits, moe_masks):
    return pl.pallas_call(
        _routing_body,
        out_shape=(
            jax.ShapeDtypeStruct((B, 2), jnp.float32),
            jax.ShapeDtypeStruct((B, 2), jnp.int32),
            jax.ShapeDtypeStruct((NT, 1), jnp.int32),
        ),
    )(logits, moe_masks)


def _scatter_body(pos_hbm, gates_hbm, rs_hbm, oi_hbm, sg_hbm,
                  pos_v, g_v, rs_v, oi_v, sg_v, sem):
    pltpu.async_copy(pos_hbm, pos_v, sem).wait()
    pltpu.async_copy(gates_hbm, g_v, sem).wait()
    zero = jnp.zeros((16,), jnp.int32)
    zerof = jnp.zeros((16,), jnp.float32)
    dump = jnp.full((16,), B, jnp.int32)
    for i in range(P // 16):
        rs_v[pl.ds(i * 16, 16)] = zero
        oi_v[pl.ds(i * 16, 16)] = dump
        sg_v[pl.ds(i * 16, 16)] = zerof
    for i in range(A // 16):
        idx = pos_v[pl.ds(i * 16, 16)]
        a = i * 16 + lax.iota(jnp.int32, 16)
        plsc.store_scatter(rs_v, [idx], a >> 1)
        plsc.store_scatter(oi_v, [idx], (a & 1) * (B + 1) + (a >> 1))
        plsc.store_scatter(sg_v, [idx], g_v[pl.ds(i * 16, 16)])
    nsub = 32
    wid = lax.axis_index("s") * 2 + lax.axis_index("c")
    per = P // nsub
    base = wid * per
    pltpu.sync_copy(rs_v.at[pl.ds(base, per)], rs_hbm.at[pl.ds(base, per)])
    pltpu.sync_copy(oi_v.at[pl.ds(base, per)], oi_hbm.at[pl.ds(base, per)])
    pltpu.sync_copy(sg_v.at[pl.ds(base, per)], sg_hbm.at[pl.ds(base, per)])


def _scatter(pos_flat, gates_flat):
    mesh = plsc.VectorSubcoreMesh(core_axis_name="c", subcore_axis_name="s")
    fn = functools.partial(
        pl.kernel,
        out_type=(
            jax.ShapeDtypeStruct((P,), jnp.int32),
            jax.ShapeDtypeStruct((P,), jnp.int32),
            jax.ShapeDtypeStruct((P,), jnp.float32),
        ),
        mesh=mesh,
        scratch_types=[
            pltpu.VMEM((A,), jnp.int32),
            pltpu.VMEM((A,), jnp.float32),
            pltpu.VMEM((P,), jnp.int32),
            pltpu.VMEM((P,), jnp.int32),
            pltpu.VMEM((P,), jnp.float32),
            pltpu.SemaphoreType.DMA,
        ],
        compiler_params=pltpu.CompilerParams(needs_layout_passes=False),
    )(_scatter_body)
    return fn(pos_flat, gates_flat)


def _mm_body(rs_ref, te_ref, oi_ref, x_ref, w_ref, b_ref, sg_ref, out_ref,
             xbuf, o_acc, stage, sems, osems):
    t = pl.program_id(0)

    def issue(tile, buf):
        for j in range(TS):
            pltpu.make_async_copy(
                x_ref.at[rs_ref[tile * TS + j]],
                xbuf.at[buf, j],
                sems.at[buf, j % 8],
            ).start()

    def drain(tile, buf):
        for j in range(TS):
            pltpu.make_async_copy(
                x_ref.at[rs_ref[tile * TS + j]],
                xbuf.at[buf, j],
                sems.at[buf, j % 8],
            ).wait()

    @pl.when(t == 0)
    def _():
        issue(0, 0)

    @pl.when(t + 1 < NT)
    def _():
        issue(t + 1, (t + 1) % 2)

    drain(t, t % 2)
    x = xbuf[t % 2].reshape(TS * L, DIN)
    e = te_ref[t]
    y = jnp.dot(x, w_ref[e], preferred_element_type=jnp.float32)
    y = y + b_ref[e][None, :]
    for j in range(TS):
        slot = t * TS + j
        rowv = sg_ref[slot] * y[j * L:(j + 1) * L, :]
        o_acc[oi_ref[slot]] = rowv.astype(jnp.bfloat16)

    @pl.when(t == NT - 1)
    def _():
        nch = B // CH
        for c in range(nch):
            o1 = o_acc[pl.ds(c * CH, CH)].astype(jnp.float32)
            o2 = o_acc[pl.ds(B + 1 + c * CH, CH)].astype(jnp.float32)
            sbuf = c % 2
            if c >= 2:
                pltpu.make_async_copy(
                    stage.at[sbuf],
                    out_ref.at[pl.ds((c - 2) * CH * L, CH * L)],
                    osems.at[sbuf],
                ).wait()
            stage[sbuf] = (o1 + o2).astype(jnp.bfloat16).reshape(CH * L, DM)
            pltpu.make_async_copy(
                stage.at[sbuf], out_ref.at[pl.ds(c * CH * L, CH * L)],
                osems.at[sbuf],
            ).start()
        for c in range(nch - 2, nch):
            pltpu.make_async_copy(
                stage.at[c % 2], out_ref.at[pl.ds(c * CH * L, CH * L)],
                osems.at[c % 2],
            ).wait()


def _moe(x, W, b, row_sample, tile_expert, oidx, sgate):
    grid_spec = pltpu.PrefetchScalarGridSpec(
        num_scalar_prefetch=3,
        grid=(NT,),
        in_specs=[
            pl.BlockSpec(memory_space=pl.ANY),
            pl.BlockSpec((E, DIN, DM), lambda t, rs, te, oi: (0, 0, 0)),
            pl.BlockSpec((E, DM), lambda t, rs, te, oi: (0, 0)),
            pl.BlockSpec(memory_space=pltpu.SMEM),
        ],
        out_specs=pl.BlockSpec(memory_space=pl.ANY),
        scratch_shapes=[
            pltpu.VMEM((2, TS, L, DIN), jnp.bfloat16),
            pltpu.VMEM((OROWS, L, DM), jnp.bfloat16),
            pltpu.VMEM((2, CH * L, DM), jnp.bfloat16),
            pltpu.SemaphoreType.DMA((2, 8)),
            pltpu.SemaphoreType.DMA((2,)),
        ],
    )
    out = pl.pallas_call(
        _mm_body,
        grid_spec=grid_spec,
        out_shape=jax.ShapeDtypeStruct((B * L, DM), jnp.bfloat16),
        compiler_params=pltpu.CompilerParams(
            vmem_limit_bytes=100 * 1024 * 1024),
    )(row_sample, tile_expert, oidx, x.astype(jnp.bfloat16),
      W.astype(jnp.bfloat16), b, sgate)
    return out.reshape(B, L, DM)


def kernel(cycle_curve_data, logits, moe_masks, W, b):
    gates, pos, tile_expert = _routing(logits, moe_masks)
    row_sample, oidx, sgate = _scatter(pos.reshape(A), gates.reshape(A))
    return _moe(cycle_curve_data, W, b, row_sample, tile_expert.reshape(NT),
                oidx, sgate)
